# Initial kernel scaffold; baseline (speedup 1.0000x reference)
import jax
import jax.numpy as jnp
from jax.experimental import pallas as pl


def kernel(hidden_states, cu_seqlens, k_cache, v_cache, cache_seqlens, max_seqlen, ln_w, ln_b, attn_w, attn_b, proj_w, proj_b):
    raise NotImplementedError("write your pallas kernel here")


# speedup vs baseline: 1.2956x; 1.2956x over previous
import numpy as np
import jax
import jax.numpy as jnp
from jax.experimental import pallas as pl
from jax.experimental.pallas import tpu as pltpu

_H = 2048
_D = 128
_NQ = 16
_NKV = 16
_B = 4
_L = 1024
_C = 512
_EPS = 1e-5
_NEG = -1e9
_T = _B * _L
_QKV = _D * (_NQ + 2 * _NKV)

_BM = 1024
_BN = 512
_NJ = _QKV // _BN
_NJP = _NJ // 3

_VMEM_LIMIT = 56 * 1024 * 1024


def _ln_qkv_kernel(x_ref, lnw_ref, lnb_ref, w_ref, b_ref,
                   q_ref, k_ref, v_ref, xln_ref):
    j = pl.program_id(1)

    @pl.when(j == 0)
    def _():
        x = x_ref[...]
        mu = jnp.mean(x, axis=1, keepdims=True)
        xc = x - mu
        var = jnp.mean(xc * xc, axis=1, keepdims=True)
        xln_ref[...] = (xc * jax.lax.rsqrt(var + _EPS)) * lnw_ref[...] + lnb_ref[...]

    y = jnp.dot(xln_ref[...], w_ref[...], preferred_element_type=jnp.float32)
    y = y + b_ref[...]

    part = j // _NJP

    @pl.when(part == 0)
    def _():
        q_ref[...] = y

    @pl.when(part == 1)
    def _():
        k_ref[...] = y

    @pl.when(part == 2)
    def _():
        v_ref[...] = y


def _ln_qkv(x, ln_w, ln_b, attn_w, attn_b):
    grid = (_T // _BM, _NJ)
    return pl.pallas_call(
        _ln_qkv_kernel,
        grid=grid,
        in_specs=[
            pl.BlockSpec((_BM, _H), lambda m, j: (m, 0)),
            pl.BlockSpec((1, _H), lambda m, j: (0, 0)),
            pl.BlockSpec((1, _H), lambda m, j: (0, 0)),
            pl.BlockSpec((_H, _BN), lambda m, j: (0, j)),
            pl.BlockSpec((1, _BN), lambda m, j: (0, j)),
        ],
        out_specs=[
            pl.BlockSpec((_BM, _BN),
                         lambda m, j: (m, jnp.minimum(j, _NJP - 1))),
            pl.BlockSpec((_BM, _BN),
                         lambda m, j: (m, jnp.clip(j - _NJP, 0, _NJP - 1))),
            pl.BlockSpec((_BM, _BN),
                         lambda m, j: (m, jnp.clip(j - 2 * _NJP, 0, _NJP - 1))),
        ],
        out_shape=[jax.ShapeDtypeStruct((_T, _H), jnp.float32)] * 3,
        scratch_shapes=[pltpu.VMEM((_BM, _H), jnp.float32)],
        compiler_params=pltpu.CompilerParams(
            dimension_semantics=("parallel", "arbitrary"),
            vmem_limit_bytes=_VMEM_LIMIT,
        ),
        name="ln_qkv",
    )(x, ln_w.reshape(1, _H), ln_b.reshape(1, _H),
      attn_w, attn_b.reshape(1, _QKV))


def _attn_kernel(q_ref, kc_ref, vc_ref, kf_ref, vf_ref, o_ref):
    scale = 1.0 / np.sqrt(_D)
    q = q_ref[0]
    dn = (((1,), (1,)), ((), ()))
    s1 = jax.lax.dot_general(q, kc_ref[0], dn,
                             preferred_element_type=jnp.float32) * scale
    s2 = jax.lax.dot_general(q, kf_ref[0], dn,
                             preferred_element_type=jnp.float32) * scale
    row = jax.lax.broadcasted_iota(jnp.int32, (_L, _L), 0)
    col = jax.lax.broadcasted_iota(jnp.int32, (_L, _L), 1)
    s2 = jnp.where(col <= row, s2, _NEG)
    m = jnp.maximum(jnp.max(s1, axis=1, keepdims=True),
                    jnp.max(s2, axis=1, keepdims=True))
    p1 = jnp.exp(s1 - m)
    p2 = jnp.exp(s2 - m)
    den = jnp.sum(p1, axis=1, keepdims=True) + jnp.sum(p2, axis=1, keepdims=True)
    o = jnp.dot(p1, vc_ref[0], preferred_element_type=jnp.float32)
    o = o + jnp.dot(p2, vf_ref[0], preferred_element_type=jnp.float32)
    o_ref[0] = o / den


def _attention(q, kc, vc, kf, vf):
    grid = (_B, _NQ)
    head_spec = pl.BlockSpec((1, _L, _D), lambda b, h: (b, 0, h))
    cache_spec = pl.BlockSpec((1, _C, _D), lambda b, h: (b, 0, h))
    return pl.pallas_call(
        _attn_kernel,
        grid=grid,
        in_specs=[head_spec, cache_spec, cache_spec, head_spec, head_spec],
        out_specs=head_spec,
        out_shape=jax.ShapeDtypeStruct((_B, _L, _H), jnp.float32),
        compiler_params=pltpu.CompilerParams(
            dimension_semantics=("parallel", "arbitrary"),
            vmem_limit_bytes=_VMEM_LIMIT,
        ),
        name="attn",
    )(q, kc, vc, kf, vf)


def _proj_kernel(x_ref, w_ref, b_ref, o_ref):
    o_ref[...] = jnp.dot(x_ref[...], w_ref[...],
                         preferred_element_type=jnp.float32) + b_ref[...]


def _proj(x, proj_w, proj_b):
    bm, bn = 1024, 1024
    grid = (_T // bm, _H // bn)
    return pl.pallas_call(
        _proj_kernel,
        grid=grid,
        in_specs=[
            pl.BlockSpec((bm, _H), lambda m, n: (m, 0)),
            pl.BlockSpec((_H, bn), lambda m, n: (0, n)),
            pl.BlockSpec((1, bn), lambda m, n: (0, n)),
        ],
        out_specs=pl.BlockSpec((bm, bn), lambda m, n: (m, n)),
        out_shape=jax.ShapeDtypeStruct((_T, _H), jnp.float32),
        compiler_params=pltpu.CompilerParams(
            dimension_semantics=("parallel", "parallel"),
            vmem_limit_bytes=_VMEM_LIMIT,
        ),
        name="out_proj",
    )(x, proj_w, proj_b.reshape(1, _H))


def kernel(hidden_states, cu_seqlens, k_cache, v_cache, cache_seqlens,
           max_seqlen, ln_w, ln_b, attn_w, attn_b, proj_w, proj_b):
    q, k, v = _ln_qkv(hidden_states, ln_w, ln_b, attn_w, attn_b)
    qb = q.reshape(_B, _L, _H)
    kb = k.reshape(_B, _L, _H)
    vb = v.reshape(_B, _L, _H)
    kc = k_cache.reshape(_B, k_cache.shape[1], _NKV * _D)
    vc = v_cache.reshape(_B, v_cache.shape[1], _NKV * _D)
    ctx = _attention(qb, kc, vc, kb, vb)
    return _proj(ctx.reshape(_T, _H), proj_w, proj_b)


# speedup vs baseline: 1.4368x; 1.1090x over previous
import numpy as np
import jax
import jax.numpy as jnp
from jax.experimental import pallas as pl
from jax.experimental.pallas import tpu as pltpu

_H = 2048
_D = 128
_NQ = 16
_NKV = 16
_B = 4
_L = 1024
_C = 512
_EPS = 1e-5
_NEG = -1e9
_T = _B * _L
_QKV = _D * (_NQ + 2 * _NKV)

_BM = 1024
_BN = 512
_NJ = _QKV // _BN
_NJP = _NJ // 3

_VMEM_LIMIT = 56 * 1024 * 1024


def _ln_qkv_kernel(x_ref, lnw_ref, lnb_ref, w_ref, b_ref,
                   q_ref, k_ref, v_ref, xln_ref):
    j = pl.program_id(1)

    @pl.when(j == 0)
    def _():
        x = x_ref[...]
        mu = jnp.mean(x, axis=1, keepdims=True)
        xc = x - mu
        var = jnp.mean(xc * xc, axis=1, keepdims=True)
        xln_ref[...] = (xc * jax.lax.rsqrt(var + _EPS)) * lnw_ref[...] + lnb_ref[...]

    y = jnp.dot(xln_ref[...], w_ref[...], preferred_element_type=jnp.float32)
    y = y + b_ref[...]

    part = j // _NJP

    @pl.when(part == 0)
    def _():
        q_ref[...] = y

    @pl.when(part == 1)
    def _():
        k_ref[...] = y

    @pl.when(part == 2)
    def _():
        v_ref[...] = y


def _ln_qkv(x, ln_w, ln_b, attn_w, attn_b):
    grid = (_T // _BM, _NJ)
    return pl.pallas_call(
        _ln_qkv_kernel,
        grid=grid,
        in_specs=[
            pl.BlockSpec((_BM, _H), lambda m, j: (m, 0)),
            pl.BlockSpec((1, _H), lambda m, j: (0, 0)),
            pl.BlockSpec((1, _H), lambda m, j: (0, 0)),
            pl.BlockSpec((_H, _BN), lambda m, j: (0, j)),
            pl.BlockSpec((1, _BN), lambda m, j: (0, j)),
        ],
        out_specs=[
            pl.BlockSpec((_BM, _BN),
                         lambda m, j: (m, jnp.minimum(j, _NJP - 1))),
            pl.BlockSpec((_BM, _BN),
                         lambda m, j: (m, jnp.clip(j - _NJP, 0, _NJP - 1))),
            pl.BlockSpec((_BM, _BN),
                         lambda m, j: (m, jnp.clip(j - 2 * _NJP, 0, _NJP - 1))),
        ],
        out_shape=[jax.ShapeDtypeStruct((_T, _H), jnp.float32)] * 3,
        scratch_shapes=[pltpu.VMEM((_BM, _H), jnp.float32)],
        compiler_params=pltpu.CompilerParams(
            dimension_semantics=("parallel", "arbitrary"),
            vmem_limit_bytes=_VMEM_LIMIT,
        ),
        name="ln_qkv",
    )(x, ln_w.reshape(1, _H), ln_b.reshape(1, _H),
      attn_w, attn_b.reshape(1, _QKV))


_CQ = 256


def _attn_kernel(q_ref, kc_ref, vc_ref, kf_ref, vf_ref, o_ref):
    scale = 1.0 / np.sqrt(_D)
    kc = kc_ref[0]
    vc = vc_ref[0]
    dn = (((1,), (1,)), ((), ()))
    for c in range(_L // _CQ):
        kw = (c + 1) * _CQ
        q = q_ref[0, c * _CQ:(c + 1) * _CQ, :] * scale
        kf = kf_ref[0, :kw, :]
        vf = vf_ref[0, :kw, :]
        s1 = jax.lax.dot_general(q, kc, dn,
                                 preferred_element_type=jnp.float32)
        s2 = jax.lax.dot_general(q, kf, dn,
                                 preferred_element_type=jnp.float32)
        row = jax.lax.broadcasted_iota(jnp.int32, (_CQ, kw), 0) + c * _CQ
        col = jax.lax.broadcasted_iota(jnp.int32, (_CQ, kw), 1)
        s2 = jnp.where(col <= row, s2, _NEG)
        m = jnp.maximum(jnp.max(s1, axis=1, keepdims=True),
                        jnp.max(s2, axis=1, keepdims=True))
        p1 = jnp.exp(s1 - m)
        p2 = jnp.exp(s2 - m)
        den = (jnp.sum(p1, axis=1, keepdims=True)
               + jnp.sum(p2, axis=1, keepdims=True))
        o = jnp.dot(p1, vc, preferred_element_type=jnp.float32)
        o = o + jnp.dot(p2, vf, preferred_element_type=jnp.float32)
        o_ref[0, c * _CQ:(c + 1) * _CQ, :] = o / den


def _attention(q, kc, vc, kf, vf):
    grid = (_B, _NQ)
    head_spec = pl.BlockSpec((1, _L, _D), lambda b, h: (b, 0, h))
    cache_spec = pl.BlockSpec((1, _C, _D), lambda b, h: (b, 0, h))
    return pl.pallas_call(
        _attn_kernel,
        grid=grid,
        in_specs=[head_spec, cache_spec, cache_spec, head_spec, head_spec],
        out_specs=head_spec,
        out_shape=jax.ShapeDtypeStruct((_B, _L, _H), jnp.float32),
        compiler_params=pltpu.CompilerParams(
            dimension_semantics=("parallel", "arbitrary"),
            vmem_limit_bytes=_VMEM_LIMIT,
        ),
        name="attn",
    )(q, kc, vc, kf, vf)


def _proj_kernel(x_ref, w_ref, b_ref, o_ref):
    o_ref[...] = jnp.dot(x_ref[...], w_ref[...],
                         preferred_element_type=jnp.float32) + b_ref[...]


def _proj(x, proj_w, proj_b):
    bm, bn = 1024, 1024
    grid = (_T // bm, _H // bn)
    return pl.pallas_call(
        _proj_kernel,
        grid=grid,
        in_specs=[
            pl.BlockSpec((bm, _H), lambda m, n: (m, 0)),
            pl.BlockSpec((_H, bn), lambda m, n: (0, n)),
            pl.BlockSpec((1, bn), lambda m, n: (0, n)),
        ],
        out_specs=pl.BlockSpec((bm, bn), lambda m, n: (m, n)),
        out_shape=jax.ShapeDtypeStruct((_T, _H), jnp.float32),
        compiler_params=pltpu.CompilerParams(
            dimension_semantics=("parallel", "arbitrary"),
            vmem_limit_bytes=_VMEM_LIMIT,
        ),
        name="out_proj",
    )(x, proj_w, proj_b.reshape(1, _H))


def kernel(hidden_states, cu_seqlens, k_cache, v_cache, cache_seqlens,
           max_seqlen, ln_w, ln_b, attn_w, attn_b, proj_w, proj_b):
    q, k, v = _ln_qkv(hidden_states, ln_w, ln_b, attn_w, attn_b)
    qb = q.reshape(_B, _L, _H)
    kb = k.reshape(_B, _L, _H)
    vb = v.reshape(_B, _L, _H)
    kc = k_cache.reshape(_B, k_cache.shape[1], _NKV * _D)
    vc = v_cache.reshape(_B, v_cache.shape[1], _NKV * _D)
    ctx = _attention(qb, kc, vc, kb, vb)
    return _proj(ctx.reshape(_T, _H), proj_w, proj_b)
